# baseline (speedup 1.0000x reference)
import functools

import jax
import jax.numpy as jnp
from jax import lax
from jax.experimental import pallas as pl
from jax.experimental.pallas import tpu as pltpu
from jax.experimental.pallas import tpu_sc as plsc

_N = 10000
_D = 128
_B = 4096
_KV = 11
_KP = 16

_info = plsc.get_sparse_core_info()
_NC = _info.num_cores
_NS = _info.num_subcores
_NW = _NC * _NS
_BW = _B // _NW
_CB = 8
_NCHUNK = _BW // _CB
_GL = _CB * _KP


def _agg_body(samp_hbm, adj_hbm, fs_hbm, feat_hbm, out_hbm,
              samp_v, fi_v, si_v, wa_v, wf_v, rows_v, out_v,
              sem_a, sem_f, sem_r):
    wid = lax.axis_index("s") * _NC + lax.axis_index("c")
    base = wid * _BW
    pltpu.sync_copy(samp_hbm.at[pl.ds(base, _BW), :], samp_v)

    def chunk(c, carry):
        for j in range(_CB):
            b = c * _CB + j
            srow = samp_v[b, :]
            node = srow[_KV - 1]
            fi_v[pl.ds(j * _KP, _KP)] = srow + node * _N
            si_v[pl.ds(j * _KP, _KP)] = srow
        cpa = pltpu.async_copy(adj_hbm.at[fi_v], wa_v, sem_a)
        cpf = pltpu.async_copy(fs_hbm.at[fi_v], wf_v, sem_f)
        cpr = pltpu.async_copy(feat_hbm.at[si_v], rows_v, sem_r)
        cpa.wait()
        cpf.wait()
        cpr.wait()
        for j in range(_CB):
            w = wa_v[pl.ds(j * _KP, _KP)] + wf_v[pl.ds(j * _KP, _KP)]
            ws = [w[k] for k in range(_KV)]
            s = ws[0]
            for k in range(1, _KV):
                s = s + ws[k]
            inv_v = 1.0 / lax.broadcast(s, (_KP,))
            for d in range(_D // 16):
                acc = ws[0] * rows_v[j * _KP, pl.ds(d * 16, 16)]
                for k in range(1, _KV):
                    acc = acc + ws[k] * rows_v[j * _KP + k, pl.ds(d * 16, 16)]
                out_v[c * _CB + j, pl.ds(d * 16, 16)] = acc * inv_v
        return carry

    lax.fori_loop(0, _NCHUNK, chunk, 0)
    pltpu.sync_copy(out_v, out_hbm.at[pl.ds(base, _BW), :])


_agg = functools.partial(
    pl.kernel,
    out_type=jax.ShapeDtypeStruct((_B, _D), jnp.float32),
    scratch_types=[
        pltpu.VMEM((_BW, _KP), jnp.int32),
        pltpu.VMEM((_GL,), jnp.int32),
        pltpu.VMEM((_GL,), jnp.int32),
        pltpu.VMEM((_GL,), jnp.float32),
        pltpu.VMEM((_GL,), jnp.float32),
        pltpu.VMEM((_GL, _D), jnp.float32),
        pltpu.VMEM((_BW, _D), jnp.float32),
        pltpu.SemaphoreType.DMA,
        pltpu.SemaphoreType.DMA,
        pltpu.SemaphoreType.DMA,
    ],
    mesh=plsc.VectorSubcoreMesh(core_axis_name="c", subcore_axis_name="s"),
)(_agg_body)


def kernel(nodes, neighbors, adj, feat_sims, feat):
    nodes = nodes.astype(jnp.int32)
    neighbors = neighbors.astype(jnp.int32)
    samp = jnp.concatenate(
        [neighbors, nodes[:, None],
         jnp.zeros((_B, _KP - _KV), jnp.int32)], axis=1)
    return _agg(samp, adj.reshape(-1), feat_sims.reshape(-1), feat)


# speedup vs baseline: 1.9860x; 1.9860x over previous
import functools

import jax
import jax.numpy as jnp
from jax import lax
from jax.experimental import pallas as pl
from jax.experimental.pallas import tpu as pltpu
from jax.experimental.pallas import tpu_sc as plsc

_N = 10000
_D = 128
_B = 4096
_KV = 11
_KP = 16

_info = plsc.get_sparse_core_info()
_NC = _info.num_cores
_NS = _info.num_subcores
_NW = _NC * _NS
_BW = _B // _NW
_CB = 32
_NCHUNK = _BW // _CB
_CROWS = _CB * _KV
_GL = _BW * _KV
_GLP = _GL + 8


def _agg_body(samp_hbm, adj_hbm, fs_hbm, feat_hbm, out_hbm,
              samp_v, fi_v, si_v, wa_v, wf_v, rows_v, out_v,
              sem_a, sem_f, sem_r0, sem_r1):
    wid = lax.axis_index("s") * _NC + lax.axis_index("c")
    base = wid * _BW
    pltpu.sync_copy(samp_hbm.at[pl.ds(base, _BW), :], samp_v)

    zeros = jnp.zeros((_KP,), jnp.int32)
    fi_v[pl.ds(_GLP - _KP, _KP)] = zeros
    si_v[pl.ds(_GLP - _KP, _KP)] = zeros

    def build(b, carry):
        srow = samp_v[b, :]
        node = srow[_KV - 1]
        fi_v[pl.ds(b * _KV, _KP)] = srow + node * _N
        si_v[pl.ds(b * _KV, _KP)] = srow
        return carry

    lax.fori_loop(0, _BW, build, 0)

    cpa = pltpu.async_copy(adj_hbm.at[fi_v], wa_v, sem_a)
    cpf = pltpu.async_copy(fs_hbm.at[fi_v], wf_v, sem_f)

    def rows_copy(c, slot):
        sem = sem_r0 if slot == 0 else sem_r1
        return pltpu.make_async_copy(
            feat_hbm.at[si_v.at[pl.ds(c * _CROWS, _CROWS)]],
            rows_v.at[pl.ds(slot * _CROWS, _CROWS), :],
            sem)

    rows_copy(0, 0).start()
    cpa.wait()
    cpf.wait()

    def chunk(c, carry):
        par = lax.rem(c, 2)

        @pl.when(c + 1 < _NCHUNK)
        def _():
            @pl.when(par == 0)
            def _():
                rows_copy(c + 1, 1).start()

            @pl.when(par == 1)
            def _():
                rows_copy(c + 1, 0).start()

        @pl.when(par == 0)
        def _():
            rows_copy(c, 0).wait()

        @pl.when(par == 1)
        def _():
            rows_copy(c, 1).wait()

        roff = par * _CROWS

        def one_row(j, carry2):
            b = c * _CB + j
            g = b * _KV
            w = wa_v[pl.ds(g, _KP)] + wf_v[pl.ds(g, _KP)]
            ws = [w[k] for k in range(_KV)]
            s = ws[0]
            for k in range(1, _KV):
                s = s + ws[k]
            inv_v = 1.0 / lax.broadcast(s, (_KP,))
            r = roff + j * _KV
            for d in range(_D // 16):
                acc = ws[0] * rows_v[r, pl.ds(d * 16, 16)]
                for k in range(1, _KV):
                    acc = acc + ws[k] * rows_v[r + k, pl.ds(d * 16, 16)]
                out_v[b, pl.ds(d * 16, 16)] = acc * inv_v
            return carry2

        lax.fori_loop(0, _CB, one_row, 0)
        return carry

    lax.fori_loop(0, _NCHUNK, chunk, 0)
    pltpu.sync_copy(out_v, out_hbm.at[pl.ds(base, _BW), :])


_agg = functools.partial(
    pl.kernel,
    out_type=jax.ShapeDtypeStruct((_B, _D), jnp.float32),
    scratch_types=[
        pltpu.VMEM((_BW, _KP), jnp.int32),
        pltpu.VMEM((_GLP,), jnp.int32),
        pltpu.VMEM((_GLP,), jnp.int32),
        pltpu.VMEM((_GLP,), jnp.float32),
        pltpu.VMEM((_GLP,), jnp.float32),
        pltpu.VMEM((2 * _CROWS, _D), jnp.float32),
        pltpu.VMEM((_BW, _D), jnp.float32),
        pltpu.SemaphoreType.DMA,
        pltpu.SemaphoreType.DMA,
        pltpu.SemaphoreType.DMA,
        pltpu.SemaphoreType.DMA,
    ],
    mesh=plsc.VectorSubcoreMesh(core_axis_name="c", subcore_axis_name="s"),
)(_agg_body)


def kernel(nodes, neighbors, adj, feat_sims, feat):
    nodes = nodes.astype(jnp.int32)
    neighbors = neighbors.astype(jnp.int32)
    samp = jnp.concatenate(
        [neighbors, nodes[:, None],
         jnp.zeros((_B, _KP - _KV), jnp.int32)], axis=1)
    return _agg(samp, adj.reshape(-1), feat_sims.reshape(-1), feat)


# speedup vs baseline: 23.0203x; 11.5911x over previous
import functools

import jax
import jax.numpy as jnp
from jax import lax
from jax.experimental import pallas as pl
from jax.experimental.pallas import tpu as pltpu
from jax.experimental.pallas import tpu_sc as plsc

_N = 10000
_D = 128
_B = 4096
_KV = 11
_KP = 16

_info = plsc.get_sparse_core_info()
_NC = _info.num_cores
_NS = _info.num_subcores
_NW = _NC * _NS
_BW = _B // _NW
_CB = 16
_NCHUNK = _BW // _CB
_CROWS = _CB * _KV
_GL = _BW * _KV
_GLP = _GL + 16
_SROWS = 2 * _CROWS


def _agg_body(samp_hbm, wa_hbm, wf_hbm, feat_hbm, out_hbm,
              samp_v, si_v, wa_v, wf_v, rows_v, out_v,
              sem_r0, sem_r1):
    wid = lax.axis_index("s") * _NC + lax.axis_index("c")
    base = wid * _BW
    pltpu.sync_copy(samp_hbm.at[pl.ds(base, _BW), :], samp_v)
    pltpu.sync_copy(wa_hbm.at[pl.ds(base * _KP, _BW * _KP)], wa_v)
    pltpu.sync_copy(wf_hbm.at[pl.ds(base * _KP, _BW * _KP)], wf_v)

    si_v[pl.ds(_GLP - _KP, _KP)] = jnp.zeros((_KP,), jnp.int32)

    def build(b, carry):
        si_v[pl.ds(b * _KV, _KP)] = samp_v[b, :]
        return carry

    lax.fori_loop(0, _BW, build, 0)

    def rows_copy(c, slot):
        return pltpu.make_async_copy(
            feat_hbm.at[si_v.at[pl.ds(c * _CROWS, _CROWS)]],
            rows_v.at[pl.ds(slot * _CROWS, _CROWS), :],
            sem_r0 if slot == 0 else sem_r1)

    rows_copy(0, 0).start()

    def chunk(c, carry):
        par = lax.rem(c, 2)

        @pl.when(c + 1 < _NCHUNK)
        def _():
            @pl.when(par == 0)
            def _():
                rows_copy(c + 1, 1).start()

            @pl.when(par == 1)
            def _():
                rows_copy(c + 1, 0).start()

        @pl.when(par == 0)
        def _():
            rows_copy(c, 0).wait()

        @pl.when(par == 1)
        def _():
            rows_copy(c, 1).wait()

        soff = par * _CROWS

        def one_row(j, carry2):
            b = c * _CB + j
            w = wa_v[pl.ds(b * _KP, _KP)] + wf_v[pl.ds(b * _KP, _KP)]
            ws = [w[k] for k in range(_KV)]
            s = ws[0]
            for k in range(1, _KV):
                s = s + ws[k]
            inv_v = 1.0 / lax.broadcast(s, (_KP,))
            r = soff + j * _KV
            for d in range(_D // 16):
                acc = ws[0] * rows_v[r, pl.ds(d * 16, 16)]
                for k in range(1, _KV):
                    acc = acc + ws[k] * rows_v[r + k, pl.ds(d * 16, 16)]
                out_v[b, pl.ds(d * 16, 16)] = acc * inv_v
            return carry2

        lax.fori_loop(0, _CB, one_row, 0)
        return carry

    lax.fori_loop(0, _NCHUNK, chunk, 0)
    pltpu.sync_copy(out_v, out_hbm.at[pl.ds(base, _BW), :])


_agg = functools.partial(
    pl.kernel,
    out_type=jax.ShapeDtypeStruct((_B, _D), jnp.float32),
    scratch_types=[
        pltpu.VMEM((_BW, _KP), jnp.int32),
        pltpu.VMEM((_GLP,), jnp.int32),
        pltpu.VMEM((_BW * _KP,), jnp.float32),
        pltpu.VMEM((_BW * _KP,), jnp.float32),
        pltpu.VMEM((_SROWS, _D), jnp.float32),
        pltpu.VMEM((_BW, _D), jnp.float32),
        pltpu.SemaphoreType.DMA,
        pltpu.SemaphoreType.DMA,
    ],
    mesh=plsc.VectorSubcoreMesh(core_axis_name="c", subcore_axis_name="s"),
)(_agg_body)


def kernel(nodes, neighbors, adj, feat_sims, feat):
    nodes = nodes.astype(jnp.int32)
    neighbors = neighbors.astype(jnp.int32)
    samp = jnp.concatenate(
        [neighbors, nodes[:, None],
         jnp.zeros((_B, _KP - _KV), jnp.int32)], axis=1)
    rows = nodes[:, None]
    wa = adj[rows, samp].reshape(-1)
    wf = feat_sims[rows, samp].reshape(-1)
    return _agg(samp, wa, wf, feat)
